# baseline (speedup 1.0000x reference)
import jax
import jax.numpy as jnp
from jax.experimental import pallas as pl
from jax.experimental.pallas import tpu as pltpu
from jax.experimental.pallas import tpu_sc as plsc

_N_CLASSES = 41
_H_TILE = 96
_RC = 8
_NTILES = 32


def _sc_wpix_body(tbl_ref, tm0_ref, w_ref, tm0_buf, w_buf, sem):
    wid = jax.lax.axis_index("s") * 2 + jax.lax.axis_index("c")
    pltpu.sync_copy(tm0_ref.at[wid], tm0_buf)
    descs = []
    for k in range(tm0_buf.shape[0]):
        descs.append(
            pltpu.async_copy(tbl_ref.at[tm0_buf.at[k]], w_buf.at[k], sem)
        )
    for d in descs:
        d.wait()
    pltpu.sync_copy(w_buf, w_ref.at[wid])


def _sc_deps_body(dp_ref, td_ref, part_ref, dp_buf, td_buf, part_buf):
    wid = jax.lax.axis_index("s") * 2 + jax.lax.axis_index("c")
    n = dp_buf.shape[0]
    pltpu.sync_copy(dp_ref.at[pl.ds(wid * n, n)], dp_buf)
    pltpu.sync_copy(td_ref.at[pl.ds(wid * n, n)], td_buf)

    def _loop(j, carry):
        sacc, nacc = carry
        a = dp_buf[pl.ds(j * 16, 16)]
        t = td_buf[pl.ds(j * 16, 16)]
        m = t != 0.0
        sacc = sacc + jnp.abs(jnp.where(m, a, 0.0) - t)
        nacc = nacc + jnp.where(m, 1.0, 0.0)
        return sacc, nacc

    z16 = jnp.zeros((16,), jnp.float32)
    sacc, nacc = jax.lax.fori_loop(0, n // 16, _loop, (z16, z16))
    part_buf[0, :] = sacc
    part_buf[1, :] = nacc
    pltpu.sync_copy(part_buf, part_ref.at[wid])


def _tc_body(x_ref, tm_ref, w_ref, out_ref):
    ht = pl.program_id(0)
    b = pl.program_id(1)

    @pl.when((ht == 0) & (b == 0))
    def _init():
        out_ref[0] = 0.0
        out_ref[1] = 0.0

    aacc = jnp.zeros((_RC, w_ref.shape[1]), jnp.float32)
    for r in range(_H_TILE // _RC):
        rows = pl.ds(r * _RC, _RC)
        tmr = tm_ref[0, rows, :].astype(jnp.int16)
        s0 = jnp.exp(x_ref[0, 0, rows, :].astype(jnp.bfloat16))
        s1 = jnp.zeros_like(s0)
        xt = jnp.zeros_like(s0)
        for c in range(1, _N_CLASSES):
            xc = x_ref[0, c, rows, :].astype(jnp.bfloat16)
            if c % 2 == 0:
                s0 = s0 + jnp.exp(xc)
            else:
                s1 = s1 + jnp.exp(xc)
            xt = jnp.where(tmr == c, xc, xt)
        s = s0 + s1
        nll = jnp.where(
            tmr != 0, jnp.log(s.astype(jnp.float32)) - xt.astype(jnp.float32), 0.0
        )
        aacc = aacc + w_ref[rows, :] * nll

    out_ref[0] = out_ref[0] + jnp.sum(aacc)
    active = tm_ref[0] != 0
    out_ref[1] = out_ref[1] + jnp.sum(active.astype(jnp.float32))


def kernel(loss_weight, masks_pred, deps_pred, true_masks, true_deps):
    B, C, H, W = masks_pred.shape
    n_ht = H // _H_TILE
    npix = H * W
    ndep = B * H * W
    pw = npix // _NTILES
    pd = ndep // _NTILES

    rows = npix // (_NTILES * 128)
    tm0_r = true_masks[0].reshape(_NTILES, rows, 128)
    dp_flat = deps_pred.reshape(ndep)
    td_flat = true_deps.reshape(ndep)
    tbl = jnp.concatenate([jnp.zeros((1,), jnp.float32), loss_weight,
                           jnp.zeros((7,), jnp.float32)])

    mesh = plsc.VectorSubcoreMesh(core_axis_name="c", subcore_axis_name="s")

    w_pix = pl.kernel(
        _sc_wpix_body,
        out_type=jax.ShapeDtypeStruct((_NTILES, rows, 128), jnp.float32),
        mesh=mesh,
        scratch_types=[
            pltpu.VMEM((rows, 128), jnp.int32),
            pltpu.VMEM((rows, 128), jnp.float32),
            pltpu.SemaphoreType.DMA,
        ],
    )(tbl, tm0_r)

    dep_part = pl.kernel(
        _sc_deps_body,
        out_type=jax.ShapeDtypeStruct((_NTILES, 2, 16), jnp.float32),
        mesh=mesh,
        scratch_types=[
            pltpu.VMEM((pd,), jnp.float32),
            pltpu.VMEM((pd,), jnp.float32),
            pltpu.VMEM((2, 16), jnp.float32),
        ],
    )(dp_flat, td_flat)

    w2d = w_pix.reshape(H, W)
    out = pl.pallas_call(
        _tc_body,
        grid=(n_ht, B),
        in_specs=[
            pl.BlockSpec((1, C, _H_TILE, W), lambda ht, b: (b, 0, ht, 0)),
            pl.BlockSpec((1, _H_TILE, W), lambda ht, b: (b, ht, 0)),
            pl.BlockSpec((_H_TILE, W), lambda ht, b: (ht, 0)),
        ],
        out_specs=pl.BlockSpec(memory_space=pltpu.SMEM),
        out_shape=jax.ShapeDtypeStruct((2,), jnp.float32),
    )(masks_pred, true_masks, w2d)

    a, n_seg = out[0], out[1]
    s_dep = jnp.sum(dep_part[:, 0, :])
    n_dep = jnp.sum(dep_part[:, 1, :])
    loss_aux = a / n_seg
    loss_main = s_dep / n_dep
    loss = loss_aux / jax.lax.stop_gradient(loss_aux) + loss_main / jax.lax.stop_gradient(loss_main)
    return loss


# speedup vs baseline: 11.5160x; 11.5160x over previous
import jax
import jax.numpy as jnp
from jax.experimental import pallas as pl
from jax.experimental.pallas import tpu as pltpu
from jax.experimental.pallas import tpu_sc as plsc

_N_CLASSES = 41
_H_TILE = 96
_RC = 8
_NTILES = 32


def _sc_wpix_body(tbl_ref, tm0_ref, w_ref, tbl_buf, tm0_buf, w_buf, sem):
    wid = jax.lax.axis_index("s") * 2 + jax.lax.axis_index("c")

    @pl.when(jax.lax.axis_index("s") == 0)
    def _stage_tbl():
        pltpu.sync_copy(tbl_ref, tbl_buf)

    pltpu.sync_copy(tm0_ref.at[wid], tm0_buf)
    plsc.subcore_barrier()
    descs = []
    for k in range(tm0_buf.shape[0]):
        descs.append(
            pltpu.async_copy(tbl_buf.at[tm0_buf.at[k]], w_buf.at[k], sem)
        )
    for d in descs:
        d.wait()
    pltpu.sync_copy(w_buf, w_ref.at[wid])


def _sc_deps_body(dp_ref, td_ref, part_ref, dp_buf, td_buf, part_buf):
    wid = jax.lax.axis_index("s") * 2 + jax.lax.axis_index("c")
    n = dp_buf.shape[0]
    pltpu.sync_copy(dp_ref.at[pl.ds(wid * n, n)], dp_buf)
    pltpu.sync_copy(td_ref.at[pl.ds(wid * n, n)], td_buf)

    def _loop(j, carry):
        sacc, nacc = carry
        a = dp_buf[pl.ds(j * 16, 16)]
        t = td_buf[pl.ds(j * 16, 16)]
        m = t != 0.0
        sacc = sacc + jnp.abs(jnp.where(m, a, 0.0) - t)
        nacc = nacc + jnp.where(m, 1.0, 0.0)
        return sacc, nacc

    z16 = jnp.zeros((16,), jnp.float32)
    sacc, nacc = jax.lax.fori_loop(0, n // 16, _loop, (z16, z16))
    part_buf[0, :] = sacc
    part_buf[1, :] = nacc
    pltpu.sync_copy(part_buf, part_ref.at[wid])


def _tc_body(x_ref, tm_ref, w_ref, out_ref):
    ht = pl.program_id(0)
    b = pl.program_id(1)

    @pl.when((ht == 0) & (b == 0))
    def _init():
        out_ref[0] = 0.0
        out_ref[1] = 0.0

    aacc = jnp.zeros((_RC, w_ref.shape[1]), jnp.float32)
    for r in range(_H_TILE // _RC):
        rows = pl.ds(r * _RC, _RC)
        tmr = tm_ref[0, rows, :].astype(jnp.int16)
        s0 = jnp.exp(x_ref[0, 0, rows, :].astype(jnp.bfloat16))
        s1 = jnp.zeros_like(s0)
        xt = jnp.zeros_like(s0)
        for c in range(1, _N_CLASSES):
            xc = x_ref[0, c, rows, :].astype(jnp.bfloat16)
            if c % 2 == 0:
                s0 = s0 + jnp.exp(xc)
            else:
                s1 = s1 + jnp.exp(xc)
            xt = jnp.where(tmr == c, xc, xt)
        s = s0 + s1
        nll = jnp.where(
            tmr != 0, jnp.log(s.astype(jnp.float32)) - xt.astype(jnp.float32), 0.0
        )
        aacc = aacc + w_ref[rows, :] * nll

    out_ref[0] = out_ref[0] + jnp.sum(aacc)
    active = tm_ref[0] != 0
    out_ref[1] = out_ref[1] + jnp.sum(active.astype(jnp.float32))


def kernel(loss_weight, masks_pred, deps_pred, true_masks, true_deps):
    B, C, H, W = masks_pred.shape
    n_ht = H // _H_TILE
    npix = H * W
    ndep = B * H * W
    pw = npix // _NTILES
    pd = ndep // _NTILES

    rows = npix // (_NTILES * 128)
    tm0_r = true_masks[0].reshape(_NTILES, rows, 128)
    dp_flat = deps_pred.reshape(ndep)
    td_flat = true_deps.reshape(ndep)
    tbl = jnp.concatenate([jnp.zeros((1,), jnp.float32), loss_weight,
                           jnp.zeros((7,), jnp.float32)])

    mesh = plsc.VectorSubcoreMesh(core_axis_name="c", subcore_axis_name="s")

    w_pix = pl.kernel(
        _sc_wpix_body,
        out_type=jax.ShapeDtypeStruct((_NTILES, rows, 128), jnp.float32),
        mesh=mesh,
        scratch_types=[
            pltpu.VMEM_SHARED((48,), jnp.float32),
            pltpu.VMEM((rows, 128), jnp.int32),
            pltpu.VMEM((rows, 128), jnp.float32),
            pltpu.SemaphoreType.DMA,
        ],
    )(tbl, tm0_r)

    dep_part = pl.kernel(
        _sc_deps_body,
        out_type=jax.ShapeDtypeStruct((_NTILES, 2, 16), jnp.float32),
        mesh=mesh,
        scratch_types=[
            pltpu.VMEM((pd,), jnp.float32),
            pltpu.VMEM((pd,), jnp.float32),
            pltpu.VMEM((2, 16), jnp.float32),
        ],
    )(dp_flat, td_flat)

    w2d = w_pix.reshape(H, W)
    out = pl.pallas_call(
        _tc_body,
        grid=(n_ht, B),
        in_specs=[
            pl.BlockSpec((1, C, _H_TILE, W), lambda ht, b: (b, 0, ht, 0)),
            pl.BlockSpec((1, _H_TILE, W), lambda ht, b: (b, ht, 0)),
            pl.BlockSpec((_H_TILE, W), lambda ht, b: (ht, 0)),
        ],
        out_specs=pl.BlockSpec(memory_space=pltpu.SMEM),
        out_shape=jax.ShapeDtypeStruct((2,), jnp.float32),
    )(masks_pred, true_masks, w2d)

    a, n_seg = out[0], out[1]
    s_dep = jnp.sum(dep_part[:, 0, :])
    n_dep = jnp.sum(dep_part[:, 1, :])
    loss_aux = a / n_seg
    loss_main = s_dep / n_dep
    loss = loss_aux / jax.lax.stop_gradient(loss_aux) + loss_main / jax.lax.stop_gradient(loss_main)
    return loss


# speedup vs baseline: 12.1292x; 1.0533x over previous
import jax
import jax.numpy as jnp
from jax.experimental import pallas as pl
from jax.experimental.pallas import tpu as pltpu
from jax.experimental.pallas import tpu_sc as plsc

_N_CLASSES = 41
_H_TILE = 96
_RC = 8
_NTILES = 32


def _sc_body(tbl_ref, tm0_ref, dp_ref, td_ref, w_ref, part_ref,
             tbl_buf, tm0_buf, w_buf, dp_buf, td_buf, part_buf,
             sem, semd0, semd1):
    wid = jax.lax.axis_index("s") * 2 + jax.lax.axis_index("c")
    n = dp_buf.shape[0]
    cpd0 = pltpu.async_copy(dp_ref.at[pl.ds(wid * n, n)], dp_buf, semd0)
    cpd0.start()
    cpd1 = pltpu.async_copy(td_ref.at[pl.ds(wid * n, n)], td_buf, semd1)
    cpd1.start()

    @pl.when(jax.lax.axis_index("s") == 0)
    def _stage_tbl():
        pltpu.sync_copy(tbl_ref, tbl_buf)

    pltpu.sync_copy(tm0_ref.at[wid], tm0_buf)
    plsc.subcore_barrier()
    descs = []
    for k in range(tm0_buf.shape[0]):
        descs.append(
            pltpu.async_copy(tbl_buf.at[tm0_buf.at[k]], w_buf.at[k], sem)
        )
    for d in descs:
        d.wait()
    pltpu.sync_copy(w_buf, w_ref.at[wid])

    cpd0.wait()
    cpd1.wait()

    def _loop(j, carry):
        sacc, nacc = carry
        a = dp_buf[pl.ds(j * 16, 16)]
        t = td_buf[pl.ds(j * 16, 16)]
        m = t != 0.0
        sacc = sacc + jnp.abs(jnp.where(m, a, 0.0) - t)
        nacc = nacc + jnp.where(m, 1.0, 0.0)
        return sacc, nacc

    z16 = jnp.zeros((16,), jnp.float32)
    sacc, nacc = jax.lax.fori_loop(0, n // 16, _loop, (z16, z16), unroll=4)
    part_buf[0, :] = sacc
    part_buf[1, :] = nacc
    pltpu.sync_copy(part_buf, part_ref.at[wid])


def _tc_body(x_ref, tm_ref, w_ref, out_ref):
    ht = pl.program_id(0)
    b = pl.program_id(1)

    @pl.when((ht == 0) & (b == 0))
    def _init():
        out_ref[0] = 0.0
        out_ref[1] = 0.0

    aacc = jnp.zeros((_RC, w_ref.shape[1]), jnp.float32)
    for r in range(_H_TILE // _RC):
        rows = pl.ds(r * _RC, _RC)
        tmr = tm_ref[0, rows, :].astype(jnp.int16)
        s0 = jnp.exp(x_ref[0, 0, rows, :].astype(jnp.bfloat16))
        s1 = jnp.zeros_like(s0)
        xt = jnp.zeros_like(s0)
        for c in range(1, _N_CLASSES):
            xc = x_ref[0, c, rows, :].astype(jnp.bfloat16)
            if c % 2 == 0:
                s0 = s0 + jnp.exp(xc)
            else:
                s1 = s1 + jnp.exp(xc)
            xt = jnp.where(tmr == c, xc, xt)
        s = s0 + s1
        nll = jnp.where(
            tmr != 0, jnp.log(s.astype(jnp.float32)) - xt.astype(jnp.float32), 0.0
        )
        aacc = aacc + w_ref[rows, :] * nll

    out_ref[0] = out_ref[0] + jnp.sum(aacc)
    active = tm_ref[0] != 0
    out_ref[1] = out_ref[1] + jnp.sum(active.astype(jnp.float32))


def kernel(loss_weight, masks_pred, deps_pred, true_masks, true_deps):
    B, C, H, W = masks_pred.shape
    n_ht = H // _H_TILE
    npix = H * W
    ndep = B * H * W
    pw = npix // _NTILES
    pd = ndep // _NTILES

    rows = npix // (_NTILES * 128)
    tm0_r = true_masks[0].reshape(_NTILES, rows, 128)
    dp_flat = deps_pred.reshape(ndep)
    td_flat = true_deps.reshape(ndep)
    tbl = jnp.concatenate([jnp.zeros((1,), jnp.float32), loss_weight,
                           jnp.zeros((7,), jnp.float32)])

    mesh = plsc.VectorSubcoreMesh(core_axis_name="c", subcore_axis_name="s")

    w_pix, dep_part = pl.kernel(
        _sc_body,
        out_type=(
            jax.ShapeDtypeStruct((_NTILES, rows, 128), jnp.float32),
            jax.ShapeDtypeStruct((_NTILES, 2, 16), jnp.float32),
        ),
        mesh=mesh,
        scratch_types=[
            pltpu.VMEM_SHARED((48,), jnp.float32),
            pltpu.VMEM((rows, 128), jnp.int32),
            pltpu.VMEM((rows, 128), jnp.float32),
            pltpu.VMEM((pd,), jnp.float32),
            pltpu.VMEM((pd,), jnp.float32),
            pltpu.VMEM((2, 16), jnp.float32),
            pltpu.SemaphoreType.DMA,
            pltpu.SemaphoreType.DMA,
            pltpu.SemaphoreType.DMA,
        ],
    )(tbl, tm0_r, dp_flat, td_flat)

    w2d = w_pix.reshape(H, W)
    out = pl.pallas_call(
        _tc_body,
        grid=(n_ht, B),
        in_specs=[
            pl.BlockSpec((1, C, _H_TILE, W), lambda ht, b: (b, 0, ht, 0)),
            pl.BlockSpec((1, _H_TILE, W), lambda ht, b: (b, ht, 0)),
            pl.BlockSpec((_H_TILE, W), lambda ht, b: (ht, 0)),
        ],
        out_specs=pl.BlockSpec(memory_space=pltpu.SMEM),
        out_shape=jax.ShapeDtypeStruct((2,), jnp.float32),
    )(masks_pred, true_masks, w2d)

    a, n_seg = out[0], out[1]
    s_dep = jnp.sum(dep_part[:, 0, :])
    n_dep = jnp.sum(dep_part[:, 1, :])
    loss_aux = a / n_seg
    loss_main = s_dep / n_dep
    loss = loss_aux / jax.lax.stop_gradient(loss_aux) + loss_main / jax.lax.stop_gradient(loss_main)
    return loss
